# baseline (speedup 1.0000x reference)
import dataclasses
import functools

import jax
import jax.numpy as jnp
from jax import lax
from jax.experimental import pallas as pl
from jax.experimental.pallas import tpu as pltpu
from jax.experimental.pallas import tpu_sc as plsc

_NC = 2
_NS = 16
_NW = _NC * _NS
_CH = 128
_LANES = 16


def _sc_gather_rows(A128, idx_flat):
    n = idx_flat.shape[0]
    rank = _LANES
    per_w = n // _NW
    n_chunks = per_w // _CH
    mesh = plsc.VectorSubcoreMesh(core_axis_name="c", subcore_axis_name="s")
    cp = pltpu.CompilerParams()
    if "needs_layout_passes" in pltpu.CompilerParams.__dataclass_fields__:
        cp = dataclasses.replace(cp, needs_layout_passes=False)

    @functools.partial(
        pl.kernel,
        mesh=mesh,
        compiler_params=cp,
        out_type=jax.ShapeDtypeStruct((n * rank,), jnp.float32),
        scratch_types=[
            pltpu.VMEM((per_w,), jnp.int32),
            pltpu.VMEM((per_w,), jnp.int32),
            pltpu.VMEM((2, _CH, 128), jnp.float32),
            pltpu.VMEM((2, _CH * rank), jnp.float32),
            pltpu.SemaphoreType.DMA((2,)),
            pltpu.SemaphoreType.DMA((2,)),
        ],
    )
    def k(table_hbm, idx_hbm, out_hbm, q_v, loff_v, rows_v, c_v, gsem, wsem):
        wid = lax.axis_index("s") * _NC + lax.axis_index("c")
        base = wid * per_w
        pltpu.sync_copy(idx_hbm.at[pl.ds(base, per_w)], q_v)

        @pl.loop(0, per_w, step=_LANES)
        def _(j):
            v = q_v[pl.ds(j, _LANES)]
            q_v[pl.ds(j, _LANES)] = v >> 3
            loff_v[pl.ds(j, _LANES)] = (v & 7) * rank

        iota16 = lax.iota(jnp.int32, _LANES)

        def gather(c, b):
            pltpu.async_copy(
                table_hbm.at[q_v.at[pl.ds(c * _CH, _CH)]],
                rows_v.at[b],
                gsem.at[b],
            )

        for b in range(2):
            gather(b, b)

        @pl.loop(0, n_chunks, step=2)
        def _(c):
            for b in range(2):
                cc = c + b
                off = cc * _CH
                pltpu.make_async_copy(
                    table_hbm.at[q_v.at[pl.ds(off, _CH)]],
                    rows_v.at[b],
                    gsem.at[b],
                ).wait()
                @pl.when(cc >= 2)
                def _():
                    pltpu.make_async_copy(
                        c_v.at[b],
                        out_hbm.at[pl.ds((base + off - 2 * _CH) * rank,
                                         _CH * rank)],
                        wsem.at[b],
                    ).wait()

                @pl.loop(0, _CH)
                def _(i):
                    lv = plsc.load_gather(
                        loff_v, [jnp.full((_LANES,), off + i, jnp.int32)]
                    )
                    rowv = plsc.load_gather(
                        rows_v.at[b],
                        [jnp.full((_LANES,), i, jnp.int32), lv + iota16],
                    )
                    c_v[b, pl.ds(i * rank, rank)] = rowv

                pltpu.async_copy(
                    c_v.at[b],
                    out_hbm.at[pl.ds((base + off) * rank, _CH * rank)],
                    wsem.at[b],
                )

                @pl.when(cc + 2 < n_chunks)
                def _():
                    gather(cc + 2, b)

        for b in range(2):
            off = (n_chunks - 2 + b) * _CH
            pltpu.make_async_copy(
                c_v.at[b],
                out_hbm.at[pl.ds((base + off) * rank, _CH * rank)],
                wsem.at[b],
            ).wait()

    return k(A128, idx_flat)


def _mm_body(g_ref, b8_ref, o_ref):
    mb = o_ref.shape[0]
    g2 = g_ref[...].reshape(mb, 128)
    o_ref[...] = lax.dot(
        g2, b8_ref[...],
        precision=lax.Precision.DEFAULT,
        preferred_element_type=jnp.float32,
    )


def _tc_matmul_packed(g_flat, B8):
    n8 = g_flat.shape[0] // 128
    mb = 1024
    return pl.pallas_call(
        _mm_body,
        grid=(n8 // mb,),
        in_specs=[
            pl.BlockSpec((mb * 128,), lambda i: (i,)),
            pl.BlockSpec((128, 512), lambda i: (0, 0)),
        ],
        out_specs=pl.BlockSpec((mb, 512), lambda i: (i, 0)),
        out_shape=jax.ShapeDtypeStruct((n8, 512), jnp.float32),
    )(g_flat, B8)


def kernel(idx, A, B):
    batch, fields = idx.shape
    rank = A.shape[1]
    dim = B.shape[1]
    n = batch * fields

    idx_flat = idx.reshape(-1).astype(jnp.int32)
    A128 = A.reshape(A.shape[0] * rank // 128, 128)
    half = idx_flat.shape[0] // 2
    g1 = _sc_gather_rows(A128, idx_flat[:half])
    g2 = _sc_gather_rows(A128, idx_flat[half:])

    eye8 = jnp.eye(8, dtype=B.dtype)
    B8 = jnp.einsum("ge,rd->gred", eye8, B).reshape(8 * rank, 8 * dim)

    out8_1 = _tc_matmul_packed(g1, B8)
    out8_2 = _tc_matmul_packed(g2, B8)
    out8 = jnp.concatenate([out8_1, out8_2], axis=0)
    return out8.reshape(batch, fields, dim)


# speedup vs baseline: 1.0513x; 1.0513x over previous
import dataclasses
import functools

import jax
import jax.numpy as jnp
from jax import lax
from jax.experimental import pallas as pl
from jax.experimental.pallas import tpu as pltpu
from jax.experimental.pallas import tpu_sc as plsc

_NC = 2
_NS = 16
_NW = _NC * _NS
_CH = 128
_LANES = 16


def _sc_gather_rows(A128, idx_flat):
    n = idx_flat.shape[0]
    rank = _LANES
    per_w = n // _NW
    n_chunks = per_w // _CH
    mesh = plsc.VectorSubcoreMesh(core_axis_name="c", subcore_axis_name="s")
    cp = pltpu.CompilerParams()
    if "needs_layout_passes" in pltpu.CompilerParams.__dataclass_fields__:
        cp = dataclasses.replace(cp, needs_layout_passes=False)

    @functools.partial(
        pl.kernel,
        mesh=mesh,
        compiler_params=cp,
        out_type=jax.ShapeDtypeStruct((n * rank,), jnp.float32),
        scratch_types=[
            pltpu.VMEM((per_w,), jnp.int32),
            pltpu.VMEM((per_w,), jnp.int32),
            pltpu.VMEM((2, _CH, 128), jnp.float32),
            pltpu.VMEM((2, _CH * rank), jnp.float32),
            pltpu.SemaphoreType.DMA((2,)),
            pltpu.SemaphoreType.DMA((2,)),
        ],
    )
    def k(table_hbm, idx_hbm, out_hbm, q_v, loff_v, rows_v, c_v, gsem, wsem):
        wid = lax.axis_index("s") * _NC + lax.axis_index("c")
        base = wid * per_w
        pltpu.sync_copy(idx_hbm.at[pl.ds(base, per_w)], q_v)

        @pl.loop(0, per_w, step=_LANES)
        def _(j):
            v = q_v[pl.ds(j, _LANES)]
            q_v[pl.ds(j, _LANES)] = v >> 3
            loff_v[pl.ds(j, _LANES)] = (v & 7) * rank

        iota16 = lax.iota(jnp.int32, _LANES)

        def gather(c, b):
            pltpu.async_copy(
                table_hbm.at[q_v.at[pl.ds(c * _CH, _CH)]],
                rows_v.at[b],
                gsem.at[b],
            )

        for b in range(2):
            gather(b, b)

        @pl.loop(0, n_chunks, step=2)
        def _(c):
            for b in range(2):
                cc = c + b
                off = cc * _CH
                pltpu.make_async_copy(
                    table_hbm.at[q_v.at[pl.ds(off, _CH)]],
                    rows_v.at[b],
                    gsem.at[b],
                ).wait()
                @pl.when(cc >= 2)
                def _():
                    pltpu.make_async_copy(
                        c_v.at[b],
                        out_hbm.at[pl.ds((base + off - 2 * _CH) * rank,
                                         _CH * rank)],
                        wsem.at[b],
                    ).wait()

                @pl.loop(0, _CH)
                def _(i):
                    lv = plsc.load_gather(
                        loff_v, [jnp.full((_LANES,), off + i, jnp.int32)]
                    )
                    rowv = plsc.load_gather(
                        rows_v.at[b],
                        [jnp.full((_LANES,), i, jnp.int32), lv + iota16],
                    )
                    c_v[b, pl.ds(i * rank, rank)] = rowv

                pltpu.async_copy(
                    c_v.at[b],
                    out_hbm.at[pl.ds((base + off) * rank, _CH * rank)],
                    wsem.at[b],
                )

                @pl.when(cc + 2 < n_chunks)
                def _():
                    gather(cc + 2, b)

        for b in range(2):
            off = (n_chunks - 2 + b) * _CH
            pltpu.make_async_copy(
                c_v.at[b],
                out_hbm.at[pl.ds((base + off) * rank, _CH * rank)],
                wsem.at[b],
            ).wait()

    return k(A128, idx_flat)


def _mm_body(g_ref, b8_ref, o_ref):
    mb = o_ref.shape[0]
    g2 = g_ref[...].reshape(mb, 128)
    o_ref[...] = lax.dot(
        g2, b8_ref[...],
        precision=lax.Precision.DEFAULT,
        preferred_element_type=jnp.float32,
    )


def _tc_matmul_packed(g_flat, B8):
    n8 = g_flat.shape[0] // 128
    mb = 1024
    return pl.pallas_call(
        _mm_body,
        grid=(n8 // mb,),
        in_specs=[
            pl.BlockSpec((mb * 128,), lambda i: (i,)),
            pl.BlockSpec((128, 512), lambda i: (0, 0)),
        ],
        out_specs=pl.BlockSpec((mb, 512), lambda i: (i, 0)),
        out_shape=jax.ShapeDtypeStruct((n8, 512), jnp.float32),
    )(g_flat, B8)


def kernel(idx, A, B):
    batch, fields = idx.shape
    rank = A.shape[1]
    dim = B.shape[1]
    n = batch * fields

    idx_flat = idx.reshape(-1).astype(jnp.int32)
    A128 = A.reshape(A.shape[0] * rank // 128, 128)
    g_flat = _sc_gather_rows(A128, idx_flat)

    eye8 = jnp.eye(8, dtype=B.dtype)
    B8 = jnp.einsum("ge,rd->gred", eye8, B).reshape(8 * rank, 8 * dim)

    out8 = _tc_matmul_packed(g_flat, B8)
    return out8.reshape(batch, fields, dim)
